# baseline (speedup 1.0000x reference)
import functools

import jax
import jax.numpy as jnp
from jax import lax
from jax.experimental import pallas as pl
from jax.experimental.pallas import tpu as pltpu
from jax.experimental.pallas import tpu_sc as plsc

_D = 16
_C = 1024
_SUB = 128
_NBUF = 2
_BC = 2048


@functools.cache
def _make_gather(total: int, vocab_pad: int):
    info = plsc.get_sparse_core_info()
    nw = info.num_cores * info.num_subcores
    n_chunk = total // _C
    per_w = n_chunk // nw
    assert n_chunk * _C == total and per_w * nw == n_chunk
    assert per_w % _NBUF == 0
    mesh = plsc.VectorSubcoreMesh(core_axis_name="c", subcore_axis_name="s")

    @functools.partial(
        pl.kernel,
        mesh=mesh,
        compiler_params=pltpu.CompilerParams(use_tc_tiling_on_sc=False),
        out_type=jax.ShapeDtypeStruct((total, _D), jnp.float32),
        scratch_types=[
            pltpu.VMEM((_NBUF, _C), jnp.int32),
            pltpu.VMEM((_NBUF, _C, _D), jnp.float32),
            pltpu.SemaphoreType.DMA((_NBUF,)),
            pltpu.SemaphoreType.DMA((_NBUF,)),
            pltpu.SemaphoreType.DMA((_NBUF,)),
        ],
    )
    def gather_kernel(ids_hbm, table_hbm, out_hbm, idx_v, rows_v,
                      sem_i, sem_g, sem_s):
        wid = lax.axis_index("s") * info.num_cores + lax.axis_index("c")
        t0 = wid * per_w

        def start_idx_load(t, b):
            pltpu.async_copy(
                ids_hbm.at[pl.ds(t * _C, _C)], idx_v.at[b], sem_i.at[b])

        def wait_idx(b):
            pltpu.make_async_copy(
                ids_hbm.at[pl.ds(0, _C)], idx_v.at[b], sem_i.at[b]).wait()

        def fire_gathers(b):
            for j in range(_C // _SUB):
                pltpu.async_copy(
                    table_hbm.at[idx_v.at[b, pl.ds(j * _SUB, _SUB)]],
                    rows_v.at[b, pl.ds(j * _SUB, _SUB)],
                    sem_g.at[b])

        def wait_gathers(b):
            for j in range(_C // _SUB):
                pltpu.make_async_copy(
                    table_hbm.at[idx_v.at[b, pl.ds(j * _SUB, _SUB)]],
                    rows_v.at[b, pl.ds(j * _SUB, _SUB)],
                    sem_g.at[b]).wait()

        def start_store(t, b):
            pltpu.async_copy(
                rows_v.at[b], out_hbm.at[pl.ds(t * _C, _C)], sem_s.at[b])

        def wait_store(b):
            pltpu.make_async_copy(
                rows_v.at[b], out_hbm.at[pl.ds(0, _C)], sem_s.at[b]).wait()

        for b in range(_NBUF):
            start_idx_load(t0 + b, b)

        def body(k, carry):
            t = t0 + k * _NBUF

            @pl.when(k > 0)
            def _():
                for b in range(_NBUF):
                    wait_store(b)

            for b in range(_NBUF):
                wait_idx(b)
                fire_gathers(b)
            for b in range(_NBUF):
                wait_gathers(b)
                start_store(t + b, b)

                @pl.when(k + 1 < per_w // _NBUF)
                def _():
                    start_idx_load(t + b + _NBUF, b)

            return carry

        lax.fori_loop(0, per_w // _NBUF, body, 0)

        for b in range(_NBUF):
            wait_store(b)

    return gather_kernel


def _transpose_block(in_ref, out_ref):
    x = in_ref[...]
    nt = x.shape[0] // 128
    for k in range(nt):
        y = x[k * 128:(k + 1) * 128, :].T
        z = y.reshape(8, _D, 128).transpose(1, 0, 2).reshape(_D, 8 * 128)
        out_ref[(k // _D) * _D:(k // _D + 1) * _D,
                (k % _D) * 1024:(k % _D + 1) * 1024] = z


def _transpose_block_acc(in_ref, dummy_ref, out_ref):
    del dummy_ref
    _transpose_block(in_ref, out_ref)


@functools.cache
def _make_transpose(hist: int, half: int, second: bool):
    hb = 4
    batch = 16384
    row0 = half // hb if second else 0
    in_specs = [pl.BlockSpec((batch * _D * hb // 128, 128),
                             lambda h: (h, 0))]
    body = _transpose_block
    kwargs = {}
    if second:
        in_specs.append(pl.BlockSpec((8, batch), lambda h: (0, 0)))
        body = _transpose_block_acc
        kwargs["input_output_aliases"] = {1: 0}
    return pl.pallas_call(
        body,
        grid=(half // hb,),
        in_specs=in_specs,
        out_specs=pl.BlockSpec((_D * hb, batch), lambda h: (row0 + h, 0)),
        out_shape=jax.ShapeDtypeStruct((hist * _D, batch), jnp.float32),
        **kwargs,
    )


def _convert_block(in_ref, out_ref):
    x = in_ref[...]
    for t in range(16):
        b3 = x[:, t * 1024:(t + 1) * 1024].reshape(_D, 8, 128)
        out_ref[t * 128:(t + 1) * 128, :] = (
            b3.transpose(1, 0, 2).reshape(128, 128).T)


@functools.cache
def _make_convert(vocab: int, vocab_pad: int):
    return pl.pallas_call(
        _convert_block,
        grid=(vocab_pad // 16384,),
        in_specs=[pl.BlockSpec((_D, 16384), lambda c: (0, c))],
        out_specs=pl.BlockSpec((2048, 128), lambda c: (c, 0)),
        out_shape=jax.ShapeDtypeStruct((vocab_pad // 8, 128), jnp.float32),
    )


def kernel(input_ids, table):
    batch, hist = input_ids.shape
    ids_perm = (input_ids.T.astype(jnp.int32)
                .reshape(hist * batch // 1024, 8, 128)
                .swapaxes(1, 2)
                .reshape(hist * batch))
    j = ids_perm & 1023
    ids_flat = (ids_perm & ~jnp.int32(1023)) | ((j & 127) << 3) | (j >> 7)
    vocab = table.shape[0]
    vocab_pad = -(-vocab // 16384) * 16384
    table_sc = _make_convert(vocab, vocab_pad)(table.T).reshape(vocab_pad, _D)
    halft = hist * batch // 2
    rows_a = _make_gather(halft, vocab_pad)(ids_flat[:halft], table_sc)
    rows_b = _make_gather(halft, vocab_pad)(ids_flat[halft:], table_sc)
    r128 = lambda r: r.reshape(halft * _D // 128, 128)
    out_a = _make_transpose(hist, hist // 2, False)(r128(rows_a))
    out_p = _make_transpose(hist, hist // 2, True)(r128(rows_b), out_a)
    return out_p.reshape(hist, _D, batch).transpose(2, 0, 1)


# speedup vs baseline: 1.1227x; 1.1227x over previous
import functools

import jax
import jax.numpy as jnp
from jax import lax
from jax.experimental import pallas as pl
from jax.experimental.pallas import tpu as pltpu
from jax.experimental.pallas import tpu_sc as plsc

_D = 16
_C = 1024
_SUB = 128
_NBUF = 2
_BC = 2048


@functools.cache
def _make_gather(total: int, vocab_pad: int):
    info = plsc.get_sparse_core_info()
    nw = info.num_cores * info.num_subcores
    n_chunk = total // _C
    per_w = n_chunk // nw
    assert n_chunk * _C == total and per_w * nw == n_chunk
    assert per_w % _NBUF == 0
    mesh = plsc.VectorSubcoreMesh(core_axis_name="c", subcore_axis_name="s")

    @functools.partial(
        pl.kernel,
        mesh=mesh,
        compiler_params=pltpu.CompilerParams(use_tc_tiling_on_sc=False),
        out_type=jax.ShapeDtypeStruct((total, _D), jnp.float32),
        scratch_types=[
            pltpu.VMEM((_NBUF, _C), jnp.int32),
            pltpu.VMEM((_NBUF, _C, _D), jnp.float32),
            pltpu.SemaphoreType.DMA((_NBUF,)),
            pltpu.SemaphoreType.DMA((_NBUF,)),
            pltpu.SemaphoreType.DMA((_NBUF,)),
        ],
    )
    def gather_kernel(ids_hbm, table_hbm, out_hbm, idx_v, rows_v,
                      sem_i, sem_g, sem_s):
        wid = lax.axis_index("s") * info.num_cores + lax.axis_index("c")
        t0 = wid * per_w

        def start_idx_load(t, b):
            pltpu.async_copy(
                ids_hbm.at[pl.ds(t * _C, _C)], idx_v.at[b], sem_i.at[b])

        def wait_idx(b):
            pltpu.make_async_copy(
                ids_hbm.at[pl.ds(0, _C)], idx_v.at[b], sem_i.at[b]).wait()

        def fire_gathers(b):
            for j in range(_C // _SUB):
                pltpu.async_copy(
                    table_hbm.at[idx_v.at[b, pl.ds(j * _SUB, _SUB)]],
                    rows_v.at[b, pl.ds(j * _SUB, _SUB)],
                    sem_g.at[b])

        def wait_gathers(b):
            for j in range(_C // _SUB):
                pltpu.make_async_copy(
                    table_hbm.at[idx_v.at[b, pl.ds(j * _SUB, _SUB)]],
                    rows_v.at[b, pl.ds(j * _SUB, _SUB)],
                    sem_g.at[b]).wait()

        def start_store(t, b):
            pltpu.async_copy(
                rows_v.at[b], out_hbm.at[pl.ds(t * _C, _C)], sem_s.at[b])

        def wait_store(b):
            pltpu.make_async_copy(
                rows_v.at[b], out_hbm.at[pl.ds(0, _C)], sem_s.at[b]).wait()

        for b in range(_NBUF):
            start_idx_load(t0 + b, b)

        def body(k, carry):
            t = t0 + k * _NBUF

            @pl.when(k > 0)
            def _():
                for b in range(_NBUF):
                    wait_store(b)

            for b in range(_NBUF):
                wait_idx(b)
                fire_gathers(b)
            for b in range(_NBUF):
                wait_gathers(b)
                start_store(t + b, b)

                @pl.when(k + 1 < per_w // _NBUF)
                def _():
                    start_idx_load(t + b + _NBUF, b)

            return carry

        lax.fori_loop(0, per_w // _NBUF, body, 0)

        for b in range(_NBUF):
            wait_store(b)

    return gather_kernel


def _transpose_block(in_ref, out_ref):
    x = in_ref[...]
    nt = x.shape[0] // 128
    for k in range(nt):
        y = x[k * 128:(k + 1) * 128, :].T
        z = y.reshape(8, _D, 128).transpose(1, 0, 2).reshape(_D, 8 * 128)
        out_ref[(k // _D) * _D:(k // _D + 1) * _D,
                (k % _D) * 1024:(k % _D + 1) * 1024] = z


@functools.cache
def _make_transpose(hist: int, batch: int):
    hb = 8
    return pl.pallas_call(
        _transpose_block,
        grid=(hist // hb,),
        in_specs=[pl.BlockSpec((batch * _D * hb // 128, 128),
                               lambda h: (h, 0))],
        out_specs=pl.BlockSpec((_D * hb, batch), lambda h: (h, 0)),
        out_shape=jax.ShapeDtypeStruct((hist * _D, batch), jnp.float32),
    )


def _convert_block(in_ref, out_ref):
    x = in_ref[...]
    for t in range(16):
        b3 = x[:, t * 1024:(t + 1) * 1024].reshape(_D, 8, 128)
        out_ref[t * 128:(t + 1) * 128, :] = (
            b3.transpose(1, 0, 2).reshape(128, 128).T)


@functools.cache
def _make_convert(vocab: int, vocab_pad: int):
    return pl.pallas_call(
        _convert_block,
        grid=(vocab_pad // 16384,),
        in_specs=[pl.BlockSpec((_D, 16384), lambda c: (0, c))],
        out_specs=pl.BlockSpec((2048, 128), lambda c: (c, 0)),
        out_shape=jax.ShapeDtypeStruct((vocab_pad // 8, 128), jnp.float32),
    )


def kernel(input_ids, table):
    batch, hist = input_ids.shape
    ids_perm = (input_ids.T.astype(jnp.int32)
                .reshape(hist * batch // 1024, 8, 128)
                .swapaxes(1, 2)
                .reshape(hist * batch))
    j = ids_perm & 1023
    ids_flat = (ids_perm & ~jnp.int32(1023)) | ((j & 127) << 3) | (j >> 7)
    vocab = table.shape[0]
    vocab_pad = -(-vocab // 16384) * 16384
    table_sc = _make_convert(vocab, vocab_pad)(table.T).reshape(vocab_pad, _D)
    rows = _make_gather(hist * batch, vocab_pad)(ids_flat, table_sc)
    rows128 = rows.reshape(hist * batch * _D // 128, 128)
    out_p = _make_transpose(hist, batch)(rows128)
    return out_p.reshape(hist, _D, batch).transpose(2, 0, 1)


# speedup vs baseline: 1.1362x; 1.0120x over previous
import functools

import jax
import jax.numpy as jnp
from jax import lax
from jax.experimental import pallas as pl
from jax.experimental.pallas import tpu as pltpu
from jax.experimental.pallas import tpu_sc as plsc

_D = 16
_C = 2048
_SUB = 128
_NBUF = 2
_BC = 2048


@functools.cache
def _make_gather(total: int, vocab_pad: int):
    info = plsc.get_sparse_core_info()
    nw = info.num_cores * info.num_subcores
    n_chunk = total // _C
    per_w = n_chunk // nw
    assert n_chunk * _C == total and per_w * nw == n_chunk
    assert per_w % _NBUF == 0
    mesh = plsc.VectorSubcoreMesh(core_axis_name="c", subcore_axis_name="s")

    @functools.partial(
        pl.kernel,
        mesh=mesh,
        compiler_params=pltpu.CompilerParams(use_tc_tiling_on_sc=False),
        out_type=jax.ShapeDtypeStruct((total, _D), jnp.float32),
        scratch_types=[
            pltpu.VMEM((_NBUF, _C), jnp.int32),
            pltpu.VMEM((_NBUF, _C, _D), jnp.float32),
            pltpu.SemaphoreType.DMA((_NBUF,)),
            pltpu.SemaphoreType.DMA((_NBUF,)),
            pltpu.SemaphoreType.DMA((_NBUF,)),
        ],
    )
    def gather_kernel(ids_hbm, table_hbm, out_hbm, idx_v, rows_v,
                      sem_i, sem_g, sem_s):
        wid = lax.axis_index("s") * info.num_cores + lax.axis_index("c")
        t0 = wid * per_w

        def start_idx_load(t, b):
            pltpu.async_copy(
                ids_hbm.at[pl.ds(t * _C, _C)], idx_v.at[b], sem_i.at[b])

        def wait_idx(b):
            pltpu.make_async_copy(
                ids_hbm.at[pl.ds(0, _C)], idx_v.at[b], sem_i.at[b]).wait()

        def fire_gathers(b):
            for j in range(_C // _SUB):
                pltpu.async_copy(
                    table_hbm.at[idx_v.at[b, pl.ds(j * _SUB, _SUB)]],
                    rows_v.at[b, pl.ds(j * _SUB, _SUB)],
                    sem_g.at[b])

        def wait_gathers(b):
            for j in range(_C // _SUB):
                pltpu.make_async_copy(
                    table_hbm.at[idx_v.at[b, pl.ds(j * _SUB, _SUB)]],
                    rows_v.at[b, pl.ds(j * _SUB, _SUB)],
                    sem_g.at[b]).wait()

        def start_store(t, b):
            pltpu.async_copy(
                rows_v.at[b], out_hbm.at[pl.ds(t * _C, _C)], sem_s.at[b])

        def wait_store(b):
            pltpu.make_async_copy(
                rows_v.at[b], out_hbm.at[pl.ds(0, _C)], sem_s.at[b]).wait()

        for b in range(_NBUF):
            start_idx_load(t0 + b, b)

        def body(k, carry):
            t = t0 + k * _NBUF

            @pl.when(k > 0)
            def _():
                for b in range(_NBUF):
                    wait_store(b)

            for b in range(_NBUF):
                wait_idx(b)
                fire_gathers(b)
            for b in range(_NBUF):
                wait_gathers(b)
                start_store(t + b, b)

                @pl.when(k + 1 < per_w // _NBUF)
                def _():
                    start_idx_load(t + b + _NBUF, b)

            return carry

        lax.fori_loop(0, per_w // _NBUF, body, 0)

        for b in range(_NBUF):
            wait_store(b)

    return gather_kernel


def _transpose_block(in_ref, out_ref):
    x = in_ref[...]
    nt = x.shape[0] // 128
    for k in range(nt):
        y = x[k * 128:(k + 1) * 128, :].T
        z = y.reshape(8, _D, 128).transpose(1, 0, 2).reshape(_D, 8 * 128)
        out_ref[(k // _D) * _D:(k // _D + 1) * _D,
                (k % _D) * 1024:(k % _D + 1) * 1024] = z


@functools.cache
def _make_transpose(hist: int, batch: int):
    hb = 8
    return pl.pallas_call(
        _transpose_block,
        grid=(hist // hb,),
        in_specs=[pl.BlockSpec((batch * _D * hb // 128, 128),
                               lambda h: (h, 0))],
        out_specs=pl.BlockSpec((_D * hb, batch), lambda h: (h, 0)),
        out_shape=jax.ShapeDtypeStruct((hist * _D, batch), jnp.float32),
    )


def _convert_block(in_ref, out_ref):
    x = in_ref[...]
    for t in range(16):
        b3 = x[:, t * 1024:(t + 1) * 1024].reshape(_D, 8, 128)
        out_ref[t * 128:(t + 1) * 128, :] = (
            b3.transpose(1, 0, 2).reshape(128, 128).T)


@functools.cache
def _make_convert(vocab: int, vocab_pad: int):
    return pl.pallas_call(
        _convert_block,
        grid=(vocab_pad // 16384,),
        in_specs=[pl.BlockSpec((_D, 16384), lambda c: (0, c))],
        out_specs=pl.BlockSpec((2048, 128), lambda c: (c, 0)),
        out_shape=jax.ShapeDtypeStruct((vocab_pad // 8, 128), jnp.float32),
    )


def kernel(input_ids, table):
    batch, hist = input_ids.shape
    ids_perm = (input_ids.T.astype(jnp.int32)
                .reshape(hist * batch // 1024, 8, 128)
                .swapaxes(1, 2)
                .reshape(hist * batch))
    j = ids_perm & 1023
    ids_flat = (ids_perm & ~jnp.int32(1023)) | ((j & 127) << 3) | (j >> 7)
    vocab = table.shape[0]
    vocab_pad = -(-vocab // 16384) * 16384
    table_sc = _make_convert(vocab, vocab_pad)(table.T).reshape(vocab_pad, _D)
    rows = _make_gather(hist * batch, vocab_pad)(ids_flat, table_sc)
    rows128 = rows.reshape(hist * batch * _D // 128, 128)
    out_p = _make_transpose(hist, batch)(rows128)
    return out_p.reshape(hist, _D, batch).transpose(2, 0, 1)


# speedup vs baseline: 1.1367x; 1.0004x over previous
import functools

import jax
import jax.numpy as jnp
from jax import lax
from jax.experimental import pallas as pl
from jax.experimental.pallas import tpu as pltpu
from jax.experimental.pallas import tpu_sc as plsc

_D = 16
_C = 2048
_SUB = 128
_NBUF = 2


@functools.cache
def _make_gather(total: int, vocab_pad: int):
    info = plsc.get_sparse_core_info()
    nw = info.num_cores * info.num_subcores
    n_chunk = total // _C
    per_w = n_chunk // nw
    assert n_chunk * _C == total and per_w * nw == n_chunk
    assert per_w % _NBUF == 0
    mesh = plsc.VectorSubcoreMesh(core_axis_name="c", subcore_axis_name="s")

    @functools.partial(
        pl.kernel,
        mesh=mesh,
        compiler_params=pltpu.CompilerParams(use_tc_tiling_on_sc=False),
        out_type=jax.ShapeDtypeStruct((total, _D), jnp.float32),
        scratch_types=[
            pltpu.VMEM((_NBUF, _C), jnp.int32),
            pltpu.VMEM((_NBUF, _C, _D), jnp.float32),
            pltpu.SemaphoreType.DMA((_NBUF,)),
            pltpu.SemaphoreType.DMA((_NBUF,)),
            pltpu.SemaphoreType.DMA((_NBUF,)),
        ],
    )
    def gather_kernel(ids_hbm, table_hbm, out_hbm, idx_v, rows_v,
                      sem_i, sem_g, sem_s):
        wid = lax.axis_index("s") * info.num_cores + lax.axis_index("c")
        t0 = wid * per_w

        def start_idx_load(t, b):
            pltpu.async_copy(
                ids_hbm.at[pl.ds(t * _C, _C)], idx_v.at[b], sem_i.at[b])

        def wait_idx(b):
            pltpu.make_async_copy(
                ids_hbm.at[pl.ds(0, _C)], idx_v.at[b], sem_i.at[b]).wait()

        def fire_gathers(b):
            for j in range(_C // _SUB):
                pltpu.async_copy(
                    table_hbm.at[idx_v.at[b, pl.ds(j * _SUB, _SUB)]],
                    rows_v.at[b, pl.ds(j * _SUB, _SUB)],
                    sem_g.at[b])

        def wait_gathers(b):
            for j in range(_C // _SUB):
                pltpu.make_async_copy(
                    table_hbm.at[idx_v.at[b, pl.ds(j * _SUB, _SUB)]],
                    rows_v.at[b, pl.ds(j * _SUB, _SUB)],
                    sem_g.at[b]).wait()

        def start_store(t, b):
            pltpu.async_copy(
                rows_v.at[b], out_hbm.at[pl.ds(t * _C, _C)], sem_s.at[b])

        def wait_store(b):
            pltpu.make_async_copy(
                rows_v.at[b], out_hbm.at[pl.ds(0, _C)], sem_s.at[b]).wait()

        for b in range(_NBUF):
            start_idx_load(t0 + b, b)

        def body(k, carry):
            t = t0 + k * _NBUF

            @pl.when(k > 0)
            def _():
                for b in range(_NBUF):
                    wait_store(b)

            for b in range(_NBUF):
                wait_idx(b)
                fire_gathers(b)
            for b in range(_NBUF):
                wait_gathers(b)
                start_store(t + b, b)

                @pl.when(k + 1 < per_w // _NBUF)
                def _():
                    start_idx_load(t + b + _NBUF, b)

            return carry

        lax.fori_loop(0, per_w // _NBUF, body, 0)

        for b in range(_NBUF):
            wait_store(b)

    return gather_kernel


def _transpose_block(in_ref, out_ref):
    x = in_ref[...]
    nt = x.shape[0] // 128
    for k in range(nt):
        y = x[k * 128:(k + 1) * 128, :].T
        z = y.reshape(8, _D, 128).transpose(1, 0, 2).reshape(_D, 8 * 128)
        out_ref[(k // _D) * _D:(k // _D + 1) * _D,
                (k % _D) * 1024:(k % _D + 1) * 1024] = z


@functools.cache
def _make_transpose(hist: int, batch: int):
    hb = 8
    return pl.pallas_call(
        _transpose_block,
        grid=(hist // hb,),
        in_specs=[pl.BlockSpec((batch * _D * hb // 128, 128),
                               lambda h: (h, 0))],
        out_specs=pl.BlockSpec((_D * hb, batch), lambda h: (h, 0)),
        out_shape=jax.ShapeDtypeStruct((hist * _D, batch), jnp.float32),
    )


def _convert_block(in_ref, out_ref):
    x = in_ref[...]
    for t in range(16):
        b3 = x[:, t * 1024:(t + 1) * 1024].reshape(_D, 8, 128)
        out_ref[t * 128:(t + 1) * 128, :] = (
            b3.transpose(1, 0, 2).reshape(128, 128).T)


@functools.cache
def _make_convert(vocab: int, vocab_pad: int):
    return pl.pallas_call(
        _convert_block,
        grid=(vocab_pad // 16384,),
        in_specs=[pl.BlockSpec((_D, 16384), lambda c: (0, c))],
        out_specs=pl.BlockSpec((2048, 128), lambda c: (c, 0)),
        out_shape=jax.ShapeDtypeStruct((vocab_pad // 8, 128), jnp.float32),
    )


def kernel(input_ids, table):
    batch, hist = input_ids.shape
    ids_perm = (input_ids.T.astype(jnp.int32)
                .reshape(hist * batch // 1024, 8, 128)
                .swapaxes(1, 2)
                .reshape(hist * batch))
    j = ids_perm & 1023
    ids_flat = (ids_perm & ~jnp.int32(1023)) | ((j & 127) << 3) | (j >> 7)
    vocab = table.shape[0]
    vocab_pad = -(-vocab // 16384) * 16384
    table_sc = _make_convert(vocab, vocab_pad)(table.T).reshape(vocab_pad, _D)
    rows = _make_gather(hist * batch, vocab_pad)(ids_flat, table_sc)
    rows128 = rows.reshape(hist * batch * _D // 128, 128)
    out_p = _make_transpose(hist, batch)(rows128)
    return out_p.reshape(hist, _D, batch).transpose(2, 0, 1)
